# Initial kernel scaffold; baseline (speedup 1.0000x reference)
import jax
import jax.numpy as jnp
from jax.experimental import pallas as pl


def kernel(positions, edges, params):
    raise NotImplementedError("write your pallas kernel here")


# speedup vs baseline: 1.7186x; 1.7186x over previous
import functools

import jax
import jax.numpy as jnp
from jax import lax
from jax.experimental import pallas as pl
from jax.experimental.pallas import tpu as pltpu
from jax.experimental.pallas import tpu_sc as plsc

N_NODES = 50000
N_PAD = 53248
NBLK = N_PAD // 128
W_SC = 128
W_SEG = 512
N_WORKERS = 32
BR = 256

_LAYER_DIMS = [(3, 60), (60, 60), (60, 60), (60, 60), (60, 120), (120, 120),
               (120, 120), (120, 150), (150, 200), (200, 210), (210, 250),
               (250, 300), (300, 300), (300, 300), (300, 300), (300, 300),
               (300, 128)]


def _padf(n):
    return ((n + 127) // 128) * 128


@functools.cache
def _make_sc_gather(feat, n_edir):
    mesh = plsc.VectorSubcoreMesh(core_axis_name="c", subcore_axis_name="s")
    n_win = n_edir // W_SC

    @functools.partial(
        pl.kernel,
        out_type=jax.ShapeDtypeStruct((n_edir, feat), jnp.float32),
        mesh=mesh,
        scratch_types=[
            pltpu.VMEM((W_SC,), jnp.int32),
            pltpu.VMEM((W_SC, feat), jnp.float32),
            pltpu.SemaphoreType.DMA,
        ],
    )
    def sc_kernel(x_hbm, src_hbm, g_hbm, srcw, rows, sem):
        wid = lax.axis_index("c") * 16 + lax.axis_index("s")
        base_rounds = n_win // N_WORKERS
        extra = n_win - base_rounds * N_WORKERS
        n_my = base_rounds + jnp.where(wid < extra, 1, 0)

        def win_body(k, carry):
            eoff = pl.multiple_of((k * N_WORKERS + wid) * W_SC, W_SC)
            pltpu.sync_copy(src_hbm.at[pl.ds(eoff, W_SC)], srcw)
            pltpu.async_copy(x_hbm.at[srcw], rows, sem).wait()
            pltpu.sync_copy(rows, g_hbm.at[pl.ds(eoff, W_SC)])
            return carry

        lax.fori_loop(0, n_my, win_body, 0)

    return sc_kernel


def _tc_segsum(dstm, g, ws, wb, wf, nw_tot):
    feat = g.shape[1]

    def body(ws_ref, wb_ref, wf_ref, dst_ref, g_ref, o_ref):
        i = pl.program_id(0)
        base = wb_ref[i] * 128
        rows_iota = lax.broadcasted_iota(jnp.int32, (128, W_SEG), 0)
        oh = jnp.where(rows_iota == dst_ref[0] - base, 1.0, 0.0)
        part = jnp.dot(oh, g_ref[...], preferred_element_type=jnp.float32)

        @pl.when(wf_ref[i] == 1)
        def _():
            o_ref[...] = part

        @pl.when(wf_ref[i] == 0)
        def _():
            o_ref[...] += part

    grid_spec = pltpu.PrefetchScalarGridSpec(
        num_scalar_prefetch=3,
        grid=(nw_tot,),
        in_specs=[
            pl.BlockSpec((1, 1, W_SEG), lambda i, ws, wb, wf: (ws[i], 0, 0)),
            pl.BlockSpec((W_SEG, feat), lambda i, ws, wb, wf: (ws[i], 0)),
        ],
        out_specs=pl.BlockSpec((128, feat), lambda i, ws, wb, wf: (wb[i], 0)),
    )
    return pl.pallas_call(
        body,
        grid_spec=grid_spec,
        out_shape=jax.ShapeDtypeStruct(((NBLK + 1) * 128, feat), jnp.float32),
    )(ws, wb, wf, dstm, g)


def _tc_layer(x, s, w0p, w1p, ones_col):
    fin = x.shape[1]
    fout = w0p.shape[1]

    def body(x_ref, s_ref, w0_ref, w1_ref, o_ref):
        m = jnp.dot(x_ref[...], w0_ref[...], preferred_element_type=jnp.float32)
        m += jnp.dot(s_ref[...], w1_ref[...], preferred_element_type=jnp.float32)
        r = jnp.maximum(m, 0.0)
        lane = lax.broadcasted_iota(jnp.int32, (BR, fout), 1)
        o_ref[...] = jnp.where(lane == ones_col, 1.0, r)

    return pl.pallas_call(
        body,
        grid=(N_PAD // BR,),
        in_specs=[
            pl.BlockSpec((BR, fin), lambda i: (i, 0)),
            pl.BlockSpec((BR, fin), lambda i: (i, 0)),
            pl.BlockSpec((fin, fout), lambda i: (0, 0)),
            pl.BlockSpec((fin, fout), lambda i: (0, 0)),
        ],
        out_specs=pl.BlockSpec((BR, fout), lambda i: (i, 0)),
        out_shape=jax.ShapeDtypeStruct((N_PAD, fout), jnp.float32),
    )(x, s, w0p, w1p)


def _tc_final(x, s, w0p, w1p):
    fin = x.shape[1]
    fout = w0p.shape[1]

    def body(x_ref, s_ref, w0_ref, w1_ref, o_ref):
        i = pl.program_id(0)
        m = jnp.dot(x_ref[...], w0_ref[...], preferred_element_type=jnp.float32)
        m += jnp.dot(s_ref[...], w1_ref[...], preferred_element_type=jnp.float32)
        r = jnp.maximum(m, 0.0)
        row = lax.broadcasted_iota(jnp.int32, (BR, fout), 0) + i * BR
        r = jnp.where(row < N_NODES, r, 0.0)
        bmax = jnp.max(r, axis=0, keepdims=True)

        @pl.when(i == 0)
        def _():
            o_ref[...] = bmax

        @pl.when(i > 0)
        def _():
            o_ref[...] = jnp.maximum(o_ref[...], bmax)

    return pl.pallas_call(
        body,
        grid=(N_PAD // BR,),
        in_specs=[
            pl.BlockSpec((BR, fin), lambda i: (i, 0)),
            pl.BlockSpec((BR, fin), lambda i: (i, 0)),
            pl.BlockSpec((fin, fout), lambda i: (0, 0)),
            pl.BlockSpec((fin, fout), lambda i: (0, 0)),
        ],
        out_specs=pl.BlockSpec((1, fout), lambda i: (0, 0)),
        out_shape=jax.ShapeDtypeStruct((1, fout), jnp.float32),
    )(x, s, w0p, w1p)


def _window_tables(dst_s, n_edir):
    n_base = n_edir // W_SEG
    nw_tot = n_base + 3 * NBLK
    bnd = jnp.searchsorted(
        dst_s, jnp.arange(NBLK + 1, dtype=jnp.int32) * 128
    ).astype(jnp.int32)
    st = (bnd[:-1] // W_SEG) * W_SEG
    en = ((bnd[1:] + W_SEG - 1) // W_SEG) * W_SEG
    wc = jnp.maximum((en - st) // W_SEG, 1)
    cum = jnp.concatenate([jnp.zeros((1,), jnp.int32),
                           jnp.cumsum(wc, dtype=jnp.int32)])
    i = jnp.arange(nw_tot, dtype=jnp.int32)
    b = (jnp.searchsorted(cum, i, side="right") - 1).astype(jnp.int32)
    b = jnp.minimum(b, NBLK)
    valid = b < NBLK
    wi = i - cum[b]
    ws = jnp.where(valid, st[jnp.minimum(b, NBLK - 1)] // W_SEG + wi, 0)
    ws = jnp.minimum(ws, n_edir // W_SEG - 1)
    wb = jnp.where(valid, b, NBLK)
    wf = jnp.where(valid, (wi == 0), i == cum[NBLK]).astype(jnp.int32)
    return ws.astype(jnp.int32), wb, wf, nw_tot


def kernel(positions, edges, params):
    edges = edges.astype(jnp.int32)
    dst = jnp.concatenate([edges[:, 0], edges[:, 1]])
    src = jnp.concatenate([edges[:, 1], edges[:, 0]])
    n_edir = dst.shape[0]
    assert n_edir % W_SEG == 0
    dst_s, src_s = lax.sort((dst, src), num_keys=1)
    dstm = dst_s.reshape(n_edir // W_SEG, 1, W_SEG)
    ws, wb, wf, nw_tot = _window_tables(dst_s, n_edir)

    x = jnp.zeros((N_PAD, 128), jnp.float32)
    x = x.at[:N_NODES, :3].set(positions.astype(jnp.float32))
    x = x.at[:, 3].set(1.0)

    for li, ((din, dout), p) in enumerate(zip(_LAYER_DIMS, params)):
        last = li == len(_LAYER_DIMS) - 1
        fin = _padf(din + 1)
        fout = dout if last else _padf(dout + 1)

        w0p = jnp.zeros((fin, fout), jnp.float32)
        w0p = w0p.at[:din, :dout].set(p["W0"].T.astype(jnp.float32))
        w0p = w0p.at[din, :dout].set(p["b0"].astype(jnp.float32))
        w1p = jnp.zeros((fin, fout), jnp.float32)
        w1p = w1p.at[:din, :dout].set(p["W1"].T.astype(jnp.float32))
        w1p = w1p.at[din, :dout].set(p["b1"].astype(jnp.float32))

        g = _make_sc_gather(fin, n_edir)(x, src_s)
        s = _tc_segsum(dstm, g, ws, wb, wf, nw_tot)
        if last:
            return _tc_final(x, s, w0p, w1p)
        x = _tc_layer(x, s, w0p, w1p, dout)
    raise AssertionError("unreachable")


# speedup vs baseline: 1.8398x; 1.0705x over previous
import functools

import jax
import jax.numpy as jnp
from jax import lax
from jax.experimental import pallas as pl
from jax.experimental.pallas import tpu as pltpu
from jax.experimental.pallas import tpu_sc as plsc

N_NODES = 50000
N_PAD = 53248
NBLK = N_PAD // 128
W_SC = 128
W_SEG = 512
N_WORKERS = 32
BR = 256

_LAYER_DIMS = [(3, 60), (60, 60), (60, 60), (60, 60), (60, 120), (120, 120),
               (120, 120), (120, 150), (150, 200), (200, 210), (210, 250),
               (250, 300), (300, 300), (300, 300), (300, 300), (300, 300),
               (300, 128)]


def _padf(n):
    return ((n + 127) // 128) * 128


@functools.cache
def _make_sc_gather(feat, n_edir):
    mesh = plsc.VectorSubcoreMesh(core_axis_name="c", subcore_axis_name="s")
    n_win = n_edir // W_SC

    base_rounds = n_win // N_WORKERS
    extra = n_win - base_rounds * N_WORKERS

    @functools.partial(
        pl.kernel,
        out_type=jax.ShapeDtypeStruct((n_edir, feat), jnp.float32),
        mesh=mesh,
        scratch_types=[
            pltpu.VMEM((2 * W_SC,), jnp.int32),
            pltpu.VMEM((W_SC, feat), jnp.float32),
            pltpu.VMEM((W_SC, feat), jnp.float32),
            pltpu.SemaphoreType.DMA,
            pltpu.SemaphoreType.DMA,
        ],
    )
    def sc_kernel(x_hbm, src_hbm, g_hbm, srcw, rows0, rows1, sem0, sem1):
        wid = lax.axis_index("c") * 16 + lax.axis_index("s")
        start = wid * base_rounds + jnp.minimum(wid, extra)
        n_my = base_rounds + jnp.where(wid < extra, 1, 0)
        n_pairs = n_my // 2

        def pair_body(k, carry):
            eoff = pl.multiple_of((start + 2 * k) * W_SC, W_SC)
            pltpu.sync_copy(src_hbm.at[pl.ds(eoff, 2 * W_SC)], srcw)
            h0 = pltpu.async_copy(
                x_hbm.at[srcw.at[pl.ds(0, W_SC)]], rows0, sem0)
            h1 = pltpu.async_copy(
                x_hbm.at[srcw.at[pl.ds(W_SC, W_SC)]], rows1, sem1)
            h0.wait()
            pltpu.sync_copy(rows0, g_hbm.at[pl.ds(eoff, W_SC)])
            h1.wait()
            pltpu.sync_copy(rows1, g_hbm.at[pl.ds(eoff + W_SC, W_SC)])
            return carry

        lax.fori_loop(0, n_pairs, pair_body, 0)

        @pl.when(n_my % 2 == 1)
        def _():
            eoff = pl.multiple_of((start + n_my - 1) * W_SC, W_SC)
            pltpu.sync_copy(src_hbm.at[pl.ds(eoff, W_SC)],
                            srcw.at[pl.ds(0, W_SC)])
            pltpu.async_copy(
                x_hbm.at[srcw.at[pl.ds(0, W_SC)]], rows0, sem0).wait()
            pltpu.sync_copy(rows0, g_hbm.at[pl.ds(eoff, W_SC)])

    return sc_kernel


def _tc_segsum(dstm, g, ws, wb, wf, nw_tot):
    feat = g.shape[1]

    def body(ws_ref, wb_ref, wf_ref, dst_ref, g_ref, o_ref):
        i = pl.program_id(0)
        base = wb_ref[i] * 128
        rows_iota = lax.broadcasted_iota(jnp.int32, (128, W_SEG), 0)
        oh = jnp.where(rows_iota == dst_ref[0] - base, 1.0, 0.0)
        part = jnp.dot(oh, g_ref[...], preferred_element_type=jnp.float32)

        @pl.when(wf_ref[i] == 1)
        def _():
            o_ref[...] = part

        @pl.when(wf_ref[i] == 0)
        def _():
            o_ref[...] += part

    grid_spec = pltpu.PrefetchScalarGridSpec(
        num_scalar_prefetch=3,
        grid=(nw_tot,),
        in_specs=[
            pl.BlockSpec((1, 1, W_SEG), lambda i, ws, wb, wf: (ws[i], 0, 0)),
            pl.BlockSpec((W_SEG, feat), lambda i, ws, wb, wf: (ws[i], 0)),
        ],
        out_specs=pl.BlockSpec((128, feat), lambda i, ws, wb, wf: (wb[i], 0)),
    )
    return pl.pallas_call(
        body,
        grid_spec=grid_spec,
        out_shape=jax.ShapeDtypeStruct(((NBLK + 1) * 128, feat), jnp.float32),
    )(ws, wb, wf, dstm, g)


def _tc_layer(x, s, w0p, w1p, ones_col):
    fin = x.shape[1]
    fout = w0p.shape[1]

    def body(x_ref, s_ref, w0_ref, w1_ref, o_ref):
        m = jnp.dot(x_ref[...], w0_ref[...], preferred_element_type=jnp.float32)
        m += jnp.dot(s_ref[...], w1_ref[...], preferred_element_type=jnp.float32)
        r = jnp.maximum(m, 0.0)
        lane = lax.broadcasted_iota(jnp.int32, (BR, fout), 1)
        o_ref[...] = jnp.where(lane == ones_col, 1.0, r)

    return pl.pallas_call(
        body,
        grid=(N_PAD // BR,),
        in_specs=[
            pl.BlockSpec((BR, fin), lambda i: (i, 0)),
            pl.BlockSpec((BR, fin), lambda i: (i, 0)),
            pl.BlockSpec((fin, fout), lambda i: (0, 0)),
            pl.BlockSpec((fin, fout), lambda i: (0, 0)),
        ],
        out_specs=pl.BlockSpec((BR, fout), lambda i: (i, 0)),
        out_shape=jax.ShapeDtypeStruct((N_PAD, fout), jnp.float32),
    )(x, s, w0p, w1p)


def _tc_final(x, s, w0p, w1p):
    fin = x.shape[1]
    fout = w0p.shape[1]

    def body(x_ref, s_ref, w0_ref, w1_ref, o_ref):
        i = pl.program_id(0)
        m = jnp.dot(x_ref[...], w0_ref[...], preferred_element_type=jnp.float32)
        m += jnp.dot(s_ref[...], w1_ref[...], preferred_element_type=jnp.float32)
        r = jnp.maximum(m, 0.0)
        row = lax.broadcasted_iota(jnp.int32, (BR, fout), 0) + i * BR
        r = jnp.where(row < N_NODES, r, 0.0)
        bmax = jnp.max(r, axis=0, keepdims=True)

        @pl.when(i == 0)
        def _():
            o_ref[...] = bmax

        @pl.when(i > 0)
        def _():
            o_ref[...] = jnp.maximum(o_ref[...], bmax)

    return pl.pallas_call(
        body,
        grid=(N_PAD // BR,),
        in_specs=[
            pl.BlockSpec((BR, fin), lambda i: (i, 0)),
            pl.BlockSpec((BR, fin), lambda i: (i, 0)),
            pl.BlockSpec((fin, fout), lambda i: (0, 0)),
            pl.BlockSpec((fin, fout), lambda i: (0, 0)),
        ],
        out_specs=pl.BlockSpec((1, fout), lambda i: (0, 0)),
        out_shape=jax.ShapeDtypeStruct((1, fout), jnp.float32),
    )(x, s, w0p, w1p)


def _window_tables(key_s, n_edir):
    dst_s = key_s
    n_base = n_edir // W_SEG
    nw_tot = n_base + 3 * NBLK
    bnd = jnp.searchsorted(
        dst_s, jnp.arange(NBLK + 1, dtype=jnp.int32)
    ).astype(jnp.int32)
    st = (bnd[:-1] // W_SEG) * W_SEG
    en = ((bnd[1:] + W_SEG - 1) // W_SEG) * W_SEG
    wc = jnp.maximum((en - st) // W_SEG, 1)
    cum = jnp.concatenate([jnp.zeros((1,), jnp.int32),
                           jnp.cumsum(wc, dtype=jnp.int32)])
    i = jnp.arange(nw_tot, dtype=jnp.int32)
    b = (jnp.searchsorted(cum, i, side="right") - 1).astype(jnp.int32)
    b = jnp.minimum(b, NBLK)
    valid = b < NBLK
    wi = i - cum[b]
    ws = jnp.where(valid, st[jnp.minimum(b, NBLK - 1)] // W_SEG + wi, 0)
    ws = jnp.minimum(ws, n_edir // W_SEG - 1)
    wb = jnp.where(valid, b, NBLK)
    wf = jnp.where(valid, (wi == 0), i == cum[NBLK]).astype(jnp.int32)
    return ws.astype(jnp.int32), wb, wf, nw_tot


def kernel(positions, edges, params):
    edges = edges.astype(jnp.int32)
    dst = jnp.concatenate([edges[:, 0], edges[:, 1]])
    src = jnp.concatenate([edges[:, 1], edges[:, 0]])
    n_edir = dst.shape[0]
    assert n_edir % W_SEG == 0
    key = lax.shift_right_logical(dst, 7)
    key_s, dst_s, src_s = lax.sort((key, dst, src), num_keys=1)
    dstm = dst_s.reshape(n_edir // W_SEG, 1, W_SEG)
    ws, wb, wf, nw_tot = _window_tables(key_s, n_edir)

    x = jnp.zeros((N_PAD, 128), jnp.float32)
    x = x.at[:N_NODES, :3].set(positions.astype(jnp.float32))
    x = x.at[:, 3].set(1.0)

    for li, ((din, dout), p) in enumerate(zip(_LAYER_DIMS, params)):
        last = li == len(_LAYER_DIMS) - 1
        fin = _padf(din + 1)
        fout = dout if last else _padf(dout + 1)

        w0p = jnp.zeros((fin, fout), jnp.float32)
        w0p = w0p.at[:din, :dout].set(p["W0"].T.astype(jnp.float32))
        w0p = w0p.at[din, :dout].set(p["b0"].astype(jnp.float32))
        w1p = jnp.zeros((fin, fout), jnp.float32)
        w1p = w1p.at[:din, :dout].set(p["W1"].T.astype(jnp.float32))
        w1p = w1p.at[din, :dout].set(p["b1"].astype(jnp.float32))

        g = _make_sc_gather(fin, n_edir)(x, src_s)
        s = _tc_segsum(dstm, g, ws, wb, wf, nw_tot)
        if last:
            return _tc_final(x, s, w0p, w1p)
        x = _tc_layer(x, s, w0p, w1p, dout)
    raise AssertionError("unreachable")
